# Initial kernel scaffold; baseline (speedup 1.0000x reference)
import jax
import jax.numpy as jnp
from jax.experimental import pallas as pl


def kernel(src, tgt, src_embedding, tgt_embedding):
    raise NotImplementedError("write your pallas kernel here")


# speedup vs baseline: 1.0695x; 1.0695x over previous
import functools

import jax
import jax.numpy as jnp
from jax.experimental import pallas as pl
from jax.experimental.pallas import tpu as pltpu

N = 2048
K_NN = 32
K_TOP = 409
K_PAD = 512
N_SAMP = 15
S_PAD = 16
BLK = 256
BIG_I = 1 << 20


def _two_sum(a, b):
    s = a + b
    bv = s - a
    av = s - bv
    return s, (a - av) + (b - bv)


def _comp3(t0, t1, t2):
    s1, e1 = _two_sum(t0, t1)
    s2, e2 = _two_sum(s1, t2)
    tt, e3 = _two_sum(e1, e2)
    ti = jax.lax.bitcast_convert_type(tt, jnp.int32)
    even = (ti & 1) == 0
    up = (e3 > 0.0) == (tt > 0.0)
    ti2 = jnp.where(up, ti + 1, ti - 1)
    tt_odd = jnp.where((e3 != 0.0) & even,
                       jax.lax.bitcast_convert_type(ti2, jnp.float32), tt)
    return s2 + tt_odd


def _bf(v):
    return v.astype(jnp.bfloat16).astype(jnp.float32)


def _knn_body(xr_ref, xt_ref, idx_ref, work_ref):
    xb = xr_ref[...]
    xt = xt_ref[...]
    a0 = xb[:, 0:1]
    a1 = xb[:, 1:2]
    a2 = xb[:, 2:3]
    b0 = xt[0:1, :]
    b1 = xt[1:2, :]
    b2 = xt[2:3, :]
    dot = _comp3(_bf(a0) * _bf(b0), _bf(a1) * _bf(b1), _bf(a2) * _bf(b2))
    aa = (a0 * a0 + a2 * a2) + a1 * a1
    bb = (b0 * b0 + b2 * b2) + b1 * b1
    work_ref[...] = aa + bb - 2.0 * dot

    iota_r = jax.lax.broadcasted_iota(jnp.int32, (BLK, N), 1)
    iota_k = jax.lax.broadcasted_iota(jnp.int32, (BLK, K_NN), 1)

    def step(t, idxacc):
        w = work_ref[...]
        rowmin = jnp.min(w, axis=1, keepdims=True)
        cand = jnp.where(w == rowmin, iota_r, BIG_I)
        imin = jnp.min(cand, axis=1, keepdims=True)
        pick = iota_r == imin
        work_ref[...] = jnp.where(pick, jnp.float32(jnp.inf), w)
        return jnp.where(iota_k == t, imin, idxacc)

    idx_ref[...] = jax.lax.fori_loop(
        0, K_NN, step, jnp.zeros((BLK, K_NN), jnp.int32))


def _knn32(p):
    xr = p.T
    return pl.pallas_call(
        _knn_body,
        grid=(N // BLK,),
        in_specs=[pl.BlockSpec((BLK, 3), lambda i: (i, 0)),
                  pl.BlockSpec((3, N), lambda i: (0, 0))],
        out_specs=pl.BlockSpec((BLK, K_NN), lambda i: (i, 0)),
        out_shape=jax.ShapeDtypeStruct((N, K_NN), jnp.int32),
        scratch_shapes=[pltpu.VMEM((BLK, N), jnp.float32)],
    )(xr, p)


def _hmin_from_idx(p, idx):
    x = p.T
    nbr = x[idx]
    c = nbr - jnp.mean(nbr, axis=1, keepdims=True)
    cov = jnp.einsum('nki,nkj->nij', c, c) / float(K_NN)
    ev = jnp.linalg.eigvalsh(cov)
    return ev[:, 0]


def _rank_body(hc_ref, hr_ref, rcol_ref, rrow_ref):
    i = pl.program_id(0)
    hc = hc_ref[...]
    hr = hr_ref[...]
    ic = jax.lax.broadcasted_iota(jnp.int32, (BLK, 1), 0) + i * BLK
    ir = jax.lax.broadcasted_iota(jnp.int32, (1, N), 1)
    gt = (hr > hc).astype(jnp.int32)
    eq = ((hr == hc) & (ir < ic)).astype(jnp.int32)
    rcol_ref[...] = jnp.sum(gt + eq, axis=1, keepdims=True)
    gt2 = (hc > hr).astype(jnp.int32)
    eq2 = ((hc == hr) & (ic < ir)).astype(jnp.int32)
    part = jnp.sum(gt2 + eq2, axis=0, keepdims=True)

    @pl.when(i == 0)
    def _():
        rrow_ref[...] = jnp.zeros_like(rrow_ref)

    rrow_ref[...] += part


def _ranks(h_col, h_row):
    return pl.pallas_call(
        _rank_body,
        grid=(N // BLK,),
        in_specs=[pl.BlockSpec((BLK, 1), lambda i: (i, 0)),
                  pl.BlockSpec((1, N), lambda i: (0, 0))],
        out_specs=[pl.BlockSpec((BLK, 1), lambda i: (i, 0)),
                   pl.BlockSpec((1, N), lambda i: (0, 0))],
        out_shape=[jax.ShapeDtypeStruct((N, 1), jnp.int32),
                   jax.ShapeDtypeStruct((1, N), jnp.int32)],
    )(h_col, h_row)


def _row_of(col, fill):
    icol = jax.lax.broadcasted_iota(jnp.int32, (K_PAD, K_PAD), 0)
    irow = jax.lax.broadcasted_iota(jnp.int32, (K_PAD, K_PAD), 1)
    d = jnp.where(icol == irow, jnp.broadcast_to(col, (K_PAD, K_PAD)), fill)
    return jnp.min(d, axis=0, keepdims=True)


def _tail_math(rs_row, rt_col, hs_row, ht_col, ps, pt, es, et):

    r_col = jax.lax.broadcasted_iota(jnp.int32, (K_PAD, 1), 0)
    r_row = jax.lax.broadcasted_iota(jnp.int32, (1, K_PAD), 1)
    i_row_n = jax.lax.broadcasted_iota(jnp.int32, (1, N), 1)
    i_col_n = jax.lax.broadcasted_iota(jnp.int32, (N, 1), 0)

    cs = (rs_row == r_col)
    svals_col = jnp.sum(jnp.where(cs, jnp.broadcast_to(hs_row, (K_PAD, N)), 0.0),
                        axis=1, keepdims=True)
    i1_col = jnp.sum(jnp.where(cs, jnp.broadcast_to(i_row_n, (K_PAD, N)), 0),
                     axis=1, keepdims=True)
    ct = (rt_col == r_row)
    tvals_row = jnp.sum(jnp.where(ct, jnp.broadcast_to(ht_col, (N, K_PAD)), 0.0),
                        axis=0, keepdims=True)
    i2_row = jnp.sum(jnp.where(ct, jnp.broadcast_to(i_col_n, (N, K_PAD)), 0),
                     axis=0, keepdims=True)

    xx = svals_col * svals_col
    yy = tvals_row * tvals_row
    pp = (xx + yy) - 2.0 * (svals_col * tvals_row)
    validc = r_row < K_TOP
    ppm = jnp.where(validc, pp, jnp.float32(jnp.inf))
    nn_col = jnp.min(ppm, axis=1, keepdims=True)
    nn_col = jnp.where(r_col < K_TOP, nn_col, jnp.float32(jnp.inf))
    nn_idx_col = jnp.min(jnp.where(ppm == nn_col, jnp.broadcast_to(r_row, ppm.shape),
                                   BIG_I), axis=1, keepdims=True)

    nn_row = _row_of(nn_col, jnp.float32(jnp.inf))
    lt = (jnp.broadcast_to(nn_col, (K_PAD, K_PAD)) < nn_row)
    eq = (jnp.broadcast_to(nn_col, (K_PAD, K_PAD)) == nn_row) & \
        (jax.lax.broadcasted_iota(jnp.int32, (K_PAD, K_PAD), 0) <
         jax.lax.broadcasted_iota(jnp.int32, (K_PAD, K_PAD), 1))
    r1_row = jnp.sum((lt | eq).astype(jnp.int32), axis=0, keepdims=True)

    q_col = jax.lax.broadcasted_iota(jnp.int32, (S_PAD, 1), 0)
    s_sel = (jnp.broadcast_to(r1_row, (S_PAD, K_PAD)) == q_col)
    r_row_b = jnp.broadcast_to(r_row, (S_PAD, K_PAD))
    sel_col = jnp.sum(jnp.where(s_sel, r_row_b, 0), axis=1, keepdims=True)
    nn_idx_row = _row_of(nn_idx_col, BIG_I)
    tsel_col = jnp.sum(jnp.where(s_sel, jnp.broadcast_to(nn_idx_row, (S_PAD, K_PAD)), 0),
                       axis=1, keepdims=True)
    i1_row = _row_of(i1_col, BIG_I)
    it0_col = jnp.sum(jnp.where(s_sel, jnp.broadcast_to(i1_row, (S_PAD, K_PAD)), 0),
                      axis=1, keepdims=True)
    t_sel = (jnp.broadcast_to(r_row, (S_PAD, K_PAD)) == tsel_col)
    it1_col = jnp.sum(jnp.where(t_sel, jnp.broadcast_to(i2_row, (S_PAD, K_PAD)), 0),
                      axis=1, keepdims=True)

    i_row_nb = jnp.broadcast_to(i_row_n, (S_PAD, N))
    g0 = (i_row_nb == it0_col).astype(jnp.float32)
    g1 = (i_row_nb == it1_col).astype(jnp.float32)
    dimn = (((1,), (1,)), ((), ()))
    hp = jax.lax.Precision.HIGHEST
    opts = jax.lax.dot_general(ps, g0, dimn, precision=hp,
                               preferred_element_type=jnp.float32)
    optt = jax.lax.dot_general(pt, g1, dimn, precision=hp,
                               preferred_element_type=jnp.float32)
    oes = jax.lax.dot_general(es, g0, dimn, precision=hp,
                              preferred_element_type=jnp.float32)
    oet = jax.lax.dot_general(et, g1, dimn, precision=hp,
                              preferred_element_type=jnp.float32)
    return opts, optt, oes, oet, it0_col, it1_col


def _tail_body(rs_row_ref, rt_col_ref, hs_row_ref, ht_col_ref,
               ps_ref, pt_ref, es_ref, et_ref,
               opts_ref, optt_ref, oes_ref, oet_ref, oit0_ref, oit1_ref):
    outs = _tail_math(rs_row_ref[...], rt_col_ref[...], hs_row_ref[...],
                      ht_col_ref[...], ps_ref[...], pt_ref[...],
                      es_ref[...], et_ref[...])
    for ref, val in zip((opts_ref, optt_ref, oes_ref, oet_ref,
                         oit0_ref, oit1_ref), outs):
        ref[...] = val


def _tail(rs_row, rt_col, hs_row, ht_col, ps, ptg, es, et):
    D = es.shape[0]
    outs = [
        jax.ShapeDtypeStruct((8, S_PAD), jnp.float32),
        jax.ShapeDtypeStruct((8, S_PAD), jnp.float32),
        jax.ShapeDtypeStruct((D, S_PAD), jnp.float32),
        jax.ShapeDtypeStruct((D, S_PAD), jnp.float32),
        jax.ShapeDtypeStruct((S_PAD, 1), jnp.int32),
        jax.ShapeDtypeStruct((S_PAD, 1), jnp.int32),
    ]
    return pl.pallas_call(
        _tail_body,
        out_shape=outs,
    )(rs_row, rt_col, hs_row, ht_col, ps, ptg, es, et)


def kernel(src, tgt, src_embedding, tgt_embedding):
    ps3 = src[0]
    pt3 = tgt[0]
    idx_s = _knn32(ps3)
    idx_t = _knn32(pt3)
    h_src = _hmin_from_idx(ps3, idx_s)
    h_tgt = _hmin_from_idx(pt3, idx_t)

    hs_col = h_src[:, None]
    hs_row = h_src[None, :]
    ht_col = h_tgt[:, None]
    ht_row = h_tgt[None, :]
    _, rs_row = _ranks(hs_col, hs_row)
    rt_col, _ = _ranks(ht_col, ht_row)

    ps = jnp.pad(ps3, ((0, 5), (0, 0)))
    ptg = jnp.pad(pt3, ((0, 5), (0, 0)))
    es = src_embedding[0]
    et = tgt_embedding[0]

    opts, optt, oes, oet, oit0, oit1 = _tail(
        rs_row, rt_col, hs_row, ht_col, ps, ptg, es, et)

    it0 = oit0[:N_SAMP, 0]
    it1 = oit1[:N_SAMP, 0]
    return (opts[None, :3, :N_SAMP], optt[None, :3, :N_SAMP],
            oes[None, :, :N_SAMP], oet[None, :, :N_SAMP], it0, it1)


# speedup vs baseline: 3.6995x; 3.4591x over previous
import functools

import jax
import jax.numpy as jnp
from jax.experimental import pallas as pl
from jax.experimental.pallas import tpu as pltpu

N = 2048
K_NN = 32
K_TOP = 409
K_PAD = 512
N_SAMP = 15
S_PAD = 16
BLK = 256
BIG_I = 1 << 20


def _two_sum(a, b):
    s = a + b
    bv = s - a
    av = s - bv
    return s, (a - av) + (b - bv)


def _comp3(t0, t1, t2):
    s1, e1 = _two_sum(t0, t1)
    s2, e2 = _two_sum(s1, t2)
    tt, e3 = _two_sum(e1, e2)
    ti = jax.lax.bitcast_convert_type(tt, jnp.int32)
    even = (ti & 1) == 0
    up = (e3 > 0.0) == (tt > 0.0)
    ti2 = jnp.where(up, ti + 1, ti - 1)
    tt_odd = jnp.where((e3 != 0.0) & even,
                       jax.lax.bitcast_convert_type(ti2, jnp.float32), tt)
    return s2 + tt_odd


def _bf(v):
    return v.astype(jnp.bfloat16).astype(jnp.float32)


def _knn_body(xr_ref, xt_ref, idx_ref, work_ref):
    xb = xr_ref[...]
    xt = xt_ref[...]
    a0 = xb[:, 0:1]
    a1 = xb[:, 1:2]
    a2 = xb[:, 2:3]
    b0 = xt[0:1, :]
    b1 = xt[1:2, :]
    b2 = xt[2:3, :]
    dot = _comp3(_bf(a0) * _bf(b0), _bf(a1) * _bf(b1), _bf(a2) * _bf(b2))
    aa = (a0 * a0 + a2 * a2) + a1 * a1
    bb = (b0 * b0 + b2 * b2) + b1 * b1
    work_ref[...] = aa + bb - 2.0 * dot

    iota_r = jax.lax.broadcasted_iota(jnp.int32, (BLK, N), 1)
    iota_k = jax.lax.broadcasted_iota(jnp.int32, (BLK, K_NN), 1)

    def step(t, idxacc):
        w = work_ref[...]
        rowmin = jnp.min(w, axis=1, keepdims=True)
        cand = jnp.where(w == rowmin, iota_r, BIG_I)
        imin = jnp.min(cand, axis=1, keepdims=True)
        pick = iota_r == imin
        work_ref[...] = jnp.where(pick, jnp.float32(jnp.inf), w)
        return jnp.where(iota_k == t, imin, idxacc)

    idx_ref[...] = jax.lax.fori_loop(
        0, K_NN, step, jnp.zeros((BLK, K_NN), jnp.int32))


def _knn32(p):
    xr = p.T
    return pl.pallas_call(
        _knn_body,
        grid=(N // BLK,),
        in_specs=[pl.BlockSpec((BLK, 3), lambda i: (i, 0)),
                  pl.BlockSpec((3, N), lambda i: (0, 0))],
        out_specs=pl.BlockSpec((BLK, K_NN), lambda i: (i, 0)),
        out_shape=jax.ShapeDtypeStruct((N, K_NN), jnp.int32),
        scratch_shapes=[pltpu.VMEM((BLK, N), jnp.float32)],
    )(xr, p)


def _cov_from_idx(p, idx):
    x = p.T
    nbr = x[idx]
    c = nbr - jnp.mean(nbr, axis=1, keepdims=True)
    return jnp.einsum('nki,nkj->nij', c, c) / float(K_NN)


def _lam_body(cov_ref, lam_ref):
    cv = cov_ref[...]
    a00 = cv[:, 0:1]
    a11 = cv[:, 4:5]
    a22 = cv[:, 8:9]
    a01 = cv[:, 1:2]
    a02 = cv[:, 2:3]
    a12 = cv[:, 5:6]

    def rot(app, aqq, apq, arp, arq):
        nz = apq != 0.0
        tau = (aqq - app) / (2.0 * apq)
        tsign = jnp.where(tau >= 0.0, 1.0, -1.0)
        t = tsign / (jnp.abs(tau) + jnp.sqrt(1.0 + tau * tau))
        t = jnp.where(nz, t, 0.0)
        c = 1.0 / jnp.sqrt(1.0 + t * t)
        s = t * c
        app2 = app - t * apq
        aqq2 = aqq + t * apq
        arp2 = c * arp - s * arq
        arq2 = s * arp + c * arq
        return app2, aqq2, jnp.zeros_like(apq), arp2, arq2

    for _ in range(5):
        a00, a11, a01, a02, a12 = rot(a00, a11, a01, a02, a12)
        a00, a22, a02, a01, a12 = rot(a00, a22, a02, a01, a12)
        a11, a22, a12, a01, a02 = rot(a11, a22, a12, a01, a02)
    lam_ref[...] = jnp.minimum(jnp.minimum(a00, a11), a22)


def _lam_est(cov16):
    return pl.pallas_call(
        _lam_body,
        out_shape=jax.ShapeDtypeStruct((N, 1), jnp.float32),
    )(cov16)


def _cand_body(rl_row_ref, cov_ref, covc_ref, oidx_ref):
    rl_row = rl_row_ref[...]
    r_col = jax.lax.broadcasted_iota(jnp.int32, (K_PAD, 1), 0)
    i_row = jax.lax.broadcasted_iota(jnp.int32, (1, N), 1)
    cs = (rl_row == r_col)
    oidx_ref[...] = jnp.sum(jnp.where(cs, jnp.broadcast_to(i_row, (K_PAD, N)), 0),
                            axis=1, keepdims=True)
    covc_ref[...] = jax.lax.dot_general(
        cs.astype(jnp.float32), cov_ref[...], (((1,), (0,)), ((), ())),
        precision=jax.lax.Precision.HIGHEST,
        preferred_element_type=jnp.float32)


def _cand_select(rl_row, cov16):
    return pl.pallas_call(
        _cand_body,
        out_shape=[jax.ShapeDtypeStruct((K_PAD, 16), jnp.float32),
                   jax.ShapeDtypeStruct((K_PAD, 1), jnp.int32)],
    )(rl_row, cov16)


def _rank_body(hc_ref, hr_ref, rcol_ref, rrow_ref):
    i = pl.program_id(0)
    hc = hc_ref[...]
    hr = hr_ref[...]
    ic = jax.lax.broadcasted_iota(jnp.int32, (BLK, 1), 0) + i * BLK
    ir = jax.lax.broadcasted_iota(jnp.int32, (1, N), 1)
    gt = (hr > hc).astype(jnp.int32)
    eq = ((hr == hc) & (ir < ic)).astype(jnp.int32)
    rcol_ref[...] = jnp.sum(gt + eq, axis=1, keepdims=True)
    gt2 = (hc > hr).astype(jnp.int32)
    eq2 = ((hc == hr) & (ic < ir)).astype(jnp.int32)
    part = jnp.sum(gt2 + eq2, axis=0, keepdims=True)

    @pl.when(i == 0)
    def _():
        rrow_ref[...] = jnp.zeros_like(rrow_ref)

    rrow_ref[...] += part


def _ranks(h_col, h_row):
    return pl.pallas_call(
        _rank_body,
        grid=(N // BLK,),
        in_specs=[pl.BlockSpec((BLK, 1), lambda i: (i, 0)),
                  pl.BlockSpec((1, N), lambda i: (0, 0))],
        out_specs=[pl.BlockSpec((BLK, 1), lambda i: (i, 0)),
                   pl.BlockSpec((1, N), lambda i: (0, 0))],
        out_shape=[jax.ShapeDtypeStruct((N, 1), jnp.int32),
                   jax.ShapeDtypeStruct((1, N), jnp.int32)],
    )(h_col, h_row)


def _row_of(col, fill):
    icol = jax.lax.broadcasted_iota(jnp.int32, (K_PAD, K_PAD), 0)
    irow = jax.lax.broadcasted_iota(jnp.int32, (K_PAD, K_PAD), 1)
    d = jnp.where(icol == irow, jnp.broadcast_to(col, (K_PAD, K_PAD)), fill)
    return jnp.min(d, axis=0, keepdims=True)


def _tail_math(hs_col, hs_row, os_col, os_row, ht_col, ht_row, ot_col, ot_row,
               ps, pt, es, et):
    r_col = jax.lax.broadcasted_iota(jnp.int32, (K_PAD, 1), 0)
    r_row = jax.lax.broadcasted_iota(jnp.int32, (1, K_PAD), 1)
    i_row_n = jax.lax.broadcasted_iota(jnp.int32, (1, N), 1)

    ms = (hs_col > hs_row) | ((hs_col == hs_row) & (os_col < os_row))
    rs_row = jnp.sum(ms.astype(jnp.int32), axis=0, keepdims=True)
    mt = (ht_row > ht_col) | ((ht_row == ht_col) & (ot_row < ot_col))
    rt_col = jnp.sum(mt.astype(jnp.int32), axis=1, keepdims=True)

    cs = (rs_row == r_col)
    svals_col = jnp.sum(jnp.where(cs, jnp.broadcast_to(hs_row, cs.shape), 0.0),
                        axis=1, keepdims=True)
    i1_col = jnp.sum(jnp.where(cs, jnp.broadcast_to(os_row, cs.shape), 0),
                     axis=1, keepdims=True)
    ct = (rt_col == r_row)
    tvals_row = jnp.sum(jnp.where(ct, jnp.broadcast_to(ht_col, ct.shape), 0.0),
                        axis=0, keepdims=True)
    i2_row = jnp.sum(jnp.where(ct, jnp.broadcast_to(ot_col, ct.shape), 0),
                     axis=0, keepdims=True)

    xx = svals_col * svals_col
    yy = tvals_row * tvals_row
    pp = (xx + yy) - 2.0 * (svals_col * tvals_row)
    validc = r_row < K_TOP
    ppm = jnp.where(validc, pp, jnp.float32(jnp.inf))
    nn_col = jnp.min(ppm, axis=1, keepdims=True)
    nn_col = jnp.where(r_col < K_TOP, nn_col, jnp.float32(jnp.inf))
    nn_idx_col = jnp.min(jnp.where(ppm == nn_col, jnp.broadcast_to(r_row, ppm.shape),
                                   BIG_I), axis=1, keepdims=True)

    nn_row = _row_of(nn_col, jnp.float32(jnp.inf))
    lt = (jnp.broadcast_to(nn_col, (K_PAD, K_PAD)) < nn_row)
    eq = (jnp.broadcast_to(nn_col, (K_PAD, K_PAD)) == nn_row) & \
        (jax.lax.broadcasted_iota(jnp.int32, (K_PAD, K_PAD), 0) <
         jax.lax.broadcasted_iota(jnp.int32, (K_PAD, K_PAD), 1))
    r1_row = jnp.sum((lt | eq).astype(jnp.int32), axis=0, keepdims=True)

    q_col = jax.lax.broadcasted_iota(jnp.int32, (S_PAD, 1), 0)
    s_sel = (jnp.broadcast_to(r1_row, (S_PAD, K_PAD)) == q_col)
    r_row_b = jnp.broadcast_to(r_row, (S_PAD, K_PAD))
    sel_col = jnp.sum(jnp.where(s_sel, r_row_b, 0), axis=1, keepdims=True)
    nn_idx_row = _row_of(nn_idx_col, BIG_I)
    tsel_col = jnp.sum(jnp.where(s_sel, jnp.broadcast_to(nn_idx_row, (S_PAD, K_PAD)), 0),
                       axis=1, keepdims=True)
    i1_row = _row_of(i1_col, BIG_I)
    it0_col = jnp.sum(jnp.where(s_sel, jnp.broadcast_to(i1_row, (S_PAD, K_PAD)), 0),
                      axis=1, keepdims=True)
    t_sel = (jnp.broadcast_to(r_row, (S_PAD, K_PAD)) == tsel_col)
    it1_col = jnp.sum(jnp.where(t_sel, jnp.broadcast_to(i2_row, (S_PAD, K_PAD)), 0),
                      axis=1, keepdims=True)

    i_row_nb = jnp.broadcast_to(i_row_n, (S_PAD, N))
    g0 = (i_row_nb == it0_col).astype(jnp.float32)
    g1 = (i_row_nb == it1_col).astype(jnp.float32)
    dimn = (((1,), (1,)), ((), ()))
    hp = jax.lax.Precision.HIGHEST
    opts = jax.lax.dot_general(ps, g0, dimn, precision=hp,
                               preferred_element_type=jnp.float32)
    optt = jax.lax.dot_general(pt, g1, dimn, precision=hp,
                               preferred_element_type=jnp.float32)
    oes = jax.lax.dot_general(es, g0, dimn, precision=hp,
                              preferred_element_type=jnp.float32)
    oet = jax.lax.dot_general(et, g1, dimn, precision=hp,
                              preferred_element_type=jnp.float32)
    return opts, optt, oes, oet, it0_col, it1_col


def _tail_body(hs_col_ref, hs_row_ref, os_col_ref, os_row_ref,
               ht_col_ref, ht_row_ref, ot_col_ref, ot_row_ref,
               ps_ref, pt_ref, es_ref, et_ref,
               opts_ref, optt_ref, oes_ref, oet_ref, oit0_ref, oit1_ref):
    outs = _tail_math(hs_col_ref[...], hs_row_ref[...], os_col_ref[...],
                      os_row_ref[...], ht_col_ref[...], ht_row_ref[...],
                      ot_col_ref[...], ot_row_ref[...], ps_ref[...],
                      pt_ref[...], es_ref[...], et_ref[...])
    for ref, val in zip((opts_ref, optt_ref, oes_ref, oet_ref,
                         oit0_ref, oit1_ref), outs):
        ref[...] = val


def _tail(*args):
    D = args[10].shape[0]
    outs = [
        jax.ShapeDtypeStruct((8, S_PAD), jnp.float32),
        jax.ShapeDtypeStruct((8, S_PAD), jnp.float32),
        jax.ShapeDtypeStruct((D, S_PAD), jnp.float32),
        jax.ShapeDtypeStruct((D, S_PAD), jnp.float32),
        jax.ShapeDtypeStruct((S_PAD, 1), jnp.int32),
        jax.ShapeDtypeStruct((S_PAD, 1), jnp.int32),
    ]
    return pl.pallas_call(
        _tail_body,
        out_shape=outs,
    )(*args)


def _candidates(p3, idx):
    cov = _cov_from_idx(p3, idx)
    cov16 = jnp.pad(cov.reshape(N, 9), ((0, 0), (0, 7)))
    lam = _lam_est(cov16)
    _, rl_row = _ranks(lam, lam.reshape(1, N))
    covc, oidx = _cand_select(rl_row, cov16)
    h_cand = jnp.linalg.eigvalsh(covc[:, :9].reshape(K_PAD, 3, 3))[:, 0]
    return h_cand, oidx


def kernel(src, tgt, src_embedding, tgt_embedding):
    ps3 = src[0]
    pt3 = tgt[0]
    idx_s = _knn32(ps3)
    idx_t = _knn32(pt3)
    hs, os_col = _candidates(ps3, idx_s)
    ht, ot_col = _candidates(pt3, idx_t)

    ps = jnp.pad(ps3, ((0, 5), (0, 0)))
    ptg = jnp.pad(pt3, ((0, 5), (0, 0)))
    es = src_embedding[0]
    et = tgt_embedding[0]

    opts, optt, oes, oet, oit0, oit1 = _tail(
        hs[:, None], hs[None, :], os_col, os_col.reshape(1, K_PAD),
        ht[:, None], ht[None, :], ot_col, ot_col.reshape(1, K_PAD),
        ps, ptg, es, et)

    it0 = oit0[:N_SAMP, 0]
    it1 = oit1[:N_SAMP, 0]
    return (opts[None, :3, :N_SAMP], optt[None, :3, :N_SAMP],
            oes[None, :, :N_SAMP], oet[None, :, :N_SAMP], it0, it1)


# speedup vs baseline: 4.0380x; 1.0915x over previous
import jax
import jax.numpy as jnp
from jax.experimental import pallas as pl
from jax.experimental.pallas import tpu as pltpu

N = 2048
K_NN = 32
K_TOP = 409
K_PAD = 512
N_SAMP = 15
S_PAD = 16
BLK = 256
BIG_I = 1 << 20


def _two_sum(a, b):
    s = a + b
    bv = s - a
    av = s - bv
    return s, (a - av) + (b - bv)


def _comp3(t0, t1, t2):
    s1, e1 = _two_sum(t0, t1)
    s2, e2 = _two_sum(s1, t2)
    tt, e3 = _two_sum(e1, e2)
    ti = jax.lax.bitcast_convert_type(tt, jnp.int32)
    even = (ti & 1) == 0
    up = (e3 > 0.0) == (tt > 0.0)
    ti2 = jnp.where(up, ti + 1, ti - 1)
    tt_odd = jnp.where((e3 != 0.0) & even,
                       jax.lax.bitcast_convert_type(ti2, jnp.float32), tt)
    return s2 + tt_odd


def _bf(v):
    return v.astype(jnp.bfloat16).astype(jnp.float32)


def _knn_body(xr_ref, xt_ref, idx_ref, work_ref):
    xb = xr_ref[...]
    xt = xt_ref[...]
    a0 = xb[:, 0:1]
    a1 = xb[:, 1:2]
    a2 = xb[:, 2:3]
    b0 = xt[0:1, :]
    b1 = xt[1:2, :]
    b2 = xt[2:3, :]
    dot = _comp3(_bf(a0) * _bf(b0), _bf(a1) * _bf(b1), _bf(a2) * _bf(b2))
    aa = (a0 * a0 + a2 * a2) + a1 * a1
    bb = (b0 * b0 + b2 * b2) + b1 * b1
    work_ref[...] = aa + bb - 2.0 * dot

    iota_r = jax.lax.broadcasted_iota(jnp.int32, (BLK, N), 1)
    iota_k = jax.lax.broadcasted_iota(jnp.int32, (BLK, K_NN), 1)

    def step(t, idxacc):
        w = work_ref[...]
        rowmin = jnp.min(w, axis=1, keepdims=True)
        cand = jnp.where(w == rowmin, iota_r, BIG_I)
        imin = jnp.min(cand, axis=1, keepdims=True)
        pick = iota_r == imin
        work_ref[...] = jnp.where(pick, jnp.float32(jnp.inf), w)
        return jnp.where(iota_k == t, imin, idxacc)

    idx_ref[...] = jax.lax.fori_loop(
        0, K_NN, step, jnp.zeros((BLK, K_NN), jnp.int32))


def _knn32(p):
    xr = p.T
    return pl.pallas_call(
        _knn_body,
        grid=(N // BLK,),
        in_specs=[pl.BlockSpec((BLK, 3), lambda i: (i, 0)),
                  pl.BlockSpec((3, N), lambda i: (0, 0))],
        out_specs=pl.BlockSpec((BLK, K_NN), lambda i: (i, 0)),
        out_shape=jax.ShapeDtypeStruct((N, K_NN), jnp.int32),
        scratch_shapes=[pltpu.VMEM((BLK, N), jnp.float32)],
    )(xr, p)


def _cov_from_idx(p, idx):
    x = p.T
    nbr = x[idx]
    c = nbr - jnp.mean(nbr, axis=1, keepdims=True)
    return jnp.einsum('nki,nkj->nij', c, c) / float(K_NN)


def _lam_body(cov_ref, lam_ref):
    cv = cov_ref[...]
    a00 = cv[:, 0:1]
    a11 = cv[:, 4:5]
    a22 = cv[:, 8:9]
    a01 = cv[:, 1:2]
    a02 = cv[:, 2:3]
    a12 = cv[:, 5:6]

    def rot(app, aqq, apq, arp, arq):
        nz = apq != 0.0
        tau = (aqq - app) / (2.0 * apq)
        tsign = jnp.where(tau >= 0.0, 1.0, -1.0)
        t = tsign / (jnp.abs(tau) + jnp.sqrt(1.0 + tau * tau))
        t = jnp.where(nz, t, 0.0)
        c = 1.0 / jnp.sqrt(1.0 + t * t)
        s = t * c
        app2 = app - t * apq
        aqq2 = aqq + t * apq
        arp2 = c * arp - s * arq
        arq2 = s * arp + c * arq
        return app2, aqq2, jnp.zeros_like(apq), arp2, arq2

    for _ in range(5):
        a00, a11, a01, a02, a12 = rot(a00, a11, a01, a02, a12)
        a00, a22, a02, a01, a12 = rot(a00, a22, a02, a01, a12)
        a11, a22, a12, a01, a02 = rot(a11, a22, a12, a01, a02)
    lam_ref[...] = jnp.minimum(jnp.minimum(a00, a11), a22)


def _lam_est(cov16):
    return pl.pallas_call(
        _lam_body,
        out_shape=jax.ShapeDtypeStruct((N, 1), jnp.float32),
    )(cov16)


M_CAND = 448


def _cand_body(rl_row_ref, cov_ref, covc_ref, oidx_ref):
    rl_row = rl_row_ref[...]
    r_col = jax.lax.broadcasted_iota(jnp.int32, (M_CAND, 1), 0)
    i_row = jax.lax.broadcasted_iota(jnp.int32, (1, N), 1)
    cs = (rl_row == r_col)
    oidx_ref[...] = jnp.sum(jnp.where(cs, jnp.broadcast_to(i_row, (M_CAND, N)), 0),
                            axis=1, keepdims=True)
    covc_ref[...] = jax.lax.dot_general(
        cs.astype(jnp.float32), cov_ref[...], (((1,), (0,)), ((), ())),
        precision=jax.lax.Precision.HIGHEST,
        preferred_element_type=jnp.float32)


def _cand_select(rl_row, cov16):
    return pl.pallas_call(
        _cand_body,
        out_shape=[jax.ShapeDtypeStruct((M_CAND, 16), jnp.float32),
                   jax.ShapeDtypeStruct((M_CAND, 1), jnp.int32)],
    )(rl_row, cov16)


def _rank_body(hc_ref, hr_ref, rcol_ref, rrow_ref):
    i = pl.program_id(0)
    hc = hc_ref[...]
    hr = hr_ref[...]
    ic = jax.lax.broadcasted_iota(jnp.int32, (BLK, 1), 0) + i * BLK
    ir = jax.lax.broadcasted_iota(jnp.int32, (1, N), 1)
    gt = (hr > hc).astype(jnp.int32)
    eq = ((hr == hc) & (ir < ic)).astype(jnp.int32)
    rcol_ref[...] = jnp.sum(gt + eq, axis=1, keepdims=True)
    gt2 = (hc > hr).astype(jnp.int32)
    eq2 = ((hc == hr) & (ic < ir)).astype(jnp.int32)
    part = jnp.sum(gt2 + eq2, axis=0, keepdims=True)

    @pl.when(i == 0)
    def _():
        rrow_ref[...] = jnp.zeros_like(rrow_ref)

    rrow_ref[...] += part


def _ranks(h_col, h_row):
    return pl.pallas_call(
        _rank_body,
        grid=(N // BLK,),
        in_specs=[pl.BlockSpec((BLK, 1), lambda i: (i, 0)),
                  pl.BlockSpec((1, N), lambda i: (0, 0))],
        out_specs=[pl.BlockSpec((BLK, 1), lambda i: (i, 0)),
                   pl.BlockSpec((1, N), lambda i: (0, 0))],
        out_shape=[jax.ShapeDtypeStruct((N, 1), jnp.int32),
                   jax.ShapeDtypeStruct((1, N), jnp.int32)],
    )(h_col, h_row)


def _row_of(col, fill):
    icol = jax.lax.broadcasted_iota(jnp.int32, (K_PAD, K_PAD), 0)
    irow = jax.lax.broadcasted_iota(jnp.int32, (K_PAD, K_PAD), 1)
    d = jnp.where(icol == irow, jnp.broadcast_to(col, (K_PAD, K_PAD)), fill)
    return jnp.min(d, axis=0, keepdims=True)


def _tail_math(hs_col, hs_row, os_col, os_row, ht_col, ht_row, ot_col, ot_row,
               ps, pt, es, et):
    r_col = jax.lax.broadcasted_iota(jnp.int32, (K_PAD, 1), 0)
    r_row = jax.lax.broadcasted_iota(jnp.int32, (1, K_PAD), 1)
    i_row_n = jax.lax.broadcasted_iota(jnp.int32, (1, N), 1)

    ms = (hs_col > hs_row) | ((hs_col == hs_row) & (os_col < os_row))
    rs_row = jnp.sum(ms.astype(jnp.int32), axis=0, keepdims=True)
    mt = (ht_row > ht_col) | ((ht_row == ht_col) & (ot_row < ot_col))
    rt_col = jnp.sum(mt.astype(jnp.int32), axis=1, keepdims=True)

    cs = (rs_row == r_col)
    svals_col = jnp.sum(jnp.where(cs, jnp.broadcast_to(hs_row, cs.shape), 0.0),
                        axis=1, keepdims=True)
    i1_col = jnp.sum(jnp.where(cs, jnp.broadcast_to(os_row, cs.shape), 0),
                     axis=1, keepdims=True)
    ct = (rt_col == r_row)
    tvals_row = jnp.sum(jnp.where(ct, jnp.broadcast_to(ht_col, ct.shape), 0.0),
                        axis=0, keepdims=True)
    i2_row = jnp.sum(jnp.where(ct, jnp.broadcast_to(ot_col, ct.shape), 0),
                     axis=0, keepdims=True)

    xx = svals_col * svals_col
    yy = tvals_row * tvals_row
    pp = (xx + yy) - 2.0 * (svals_col * tvals_row)
    validc = r_row < K_TOP
    ppm = jnp.where(validc, pp, jnp.float32(jnp.inf))
    nn_col = jnp.min(ppm, axis=1, keepdims=True)
    nn_col = jnp.where(r_col < K_TOP, nn_col, jnp.float32(jnp.inf))
    nn_idx_col = jnp.min(jnp.where(ppm == nn_col, jnp.broadcast_to(r_row, ppm.shape),
                                   BIG_I), axis=1, keepdims=True)

    nn_row = _row_of(nn_col, jnp.float32(jnp.inf))
    lt = (jnp.broadcast_to(nn_col, (K_PAD, K_PAD)) < nn_row)
    eq = (jnp.broadcast_to(nn_col, (K_PAD, K_PAD)) == nn_row) & \
        (jax.lax.broadcasted_iota(jnp.int32, (K_PAD, K_PAD), 0) <
         jax.lax.broadcasted_iota(jnp.int32, (K_PAD, K_PAD), 1))
    r1_row = jnp.sum((lt | eq).astype(jnp.int32), axis=0, keepdims=True)

    q_col = jax.lax.broadcasted_iota(jnp.int32, (S_PAD, 1), 0)
    s_sel = (jnp.broadcast_to(r1_row, (S_PAD, K_PAD)) == q_col)
    r_row_b = jnp.broadcast_to(r_row, (S_PAD, K_PAD))
    sel_col = jnp.sum(jnp.where(s_sel, r_row_b, 0), axis=1, keepdims=True)
    nn_idx_row = _row_of(nn_idx_col, BIG_I)
    tsel_col = jnp.sum(jnp.where(s_sel, jnp.broadcast_to(nn_idx_row, (S_PAD, K_PAD)), 0),
                       axis=1, keepdims=True)
    i1_row = _row_of(i1_col, BIG_I)
    it0_col = jnp.sum(jnp.where(s_sel, jnp.broadcast_to(i1_row, (S_PAD, K_PAD)), 0),
                      axis=1, keepdims=True)
    t_sel = (jnp.broadcast_to(r_row, (S_PAD, K_PAD)) == tsel_col)
    it1_col = jnp.sum(jnp.where(t_sel, jnp.broadcast_to(i2_row, (S_PAD, K_PAD)), 0),
                      axis=1, keepdims=True)

    i_row_nb = jnp.broadcast_to(i_row_n, (S_PAD, N))
    g0 = (i_row_nb == it0_col).astype(jnp.float32)
    g1 = (i_row_nb == it1_col).astype(jnp.float32)
    dimn = (((1,), (1,)), ((), ()))
    hp = jax.lax.Precision.HIGHEST
    opts = jax.lax.dot_general(ps, g0, dimn, precision=hp,
                               preferred_element_type=jnp.float32)
    optt = jax.lax.dot_general(pt, g1, dimn, precision=hp,
                               preferred_element_type=jnp.float32)
    oes = jax.lax.dot_general(es, g0, dimn, precision=hp,
                              preferred_element_type=jnp.float32)
    oet = jax.lax.dot_general(et, g1, dimn, precision=hp,
                              preferred_element_type=jnp.float32)
    return opts, optt, oes, oet, it0_col, it1_col


def _tail_body(hs_col_ref, hs_row_ref, os_col_ref, os_row_ref,
               ht_col_ref, ht_row_ref, ot_col_ref, ot_row_ref,
               ps_ref, pt_ref, es_ref, et_ref,
               opts_ref, optt_ref, oes_ref, oet_ref, oit0_ref, oit1_ref):
    outs = _tail_math(hs_col_ref[...], hs_row_ref[...], os_col_ref[...],
                      os_row_ref[...], ht_col_ref[...], ht_row_ref[...],
                      ot_col_ref[...], ot_row_ref[...], ps_ref[...],
                      pt_ref[...], es_ref[...], et_ref[...])
    for ref, val in zip((opts_ref, optt_ref, oes_ref, oet_ref,
                         oit0_ref, oit1_ref), outs):
        ref[...] = val


def _tail(*args):
    D = args[10].shape[0]
    outs = [
        jax.ShapeDtypeStruct((8, S_PAD), jnp.float32),
        jax.ShapeDtypeStruct((8, S_PAD), jnp.float32),
        jax.ShapeDtypeStruct((D, S_PAD), jnp.float32),
        jax.ShapeDtypeStruct((D, S_PAD), jnp.float32),
        jax.ShapeDtypeStruct((S_PAD, 1), jnp.int32),
        jax.ShapeDtypeStruct((S_PAD, 1), jnp.int32),
    ]
    return pl.pallas_call(
        _tail_body,
        out_shape=outs,
    )(*args)


def _candidates(p3, idx):
    cov = _cov_from_idx(p3, idx)
    cov16 = jnp.pad(cov.reshape(N, 9), ((0, 0), (0, 7)))
    lam = _lam_est(cov16)
    _, rl_row = _ranks(lam, lam.reshape(1, N))
    covc, oidx = _cand_select(rl_row, cov16)
    h_cand = jnp.linalg.eigvalsh(covc[:, :9].reshape(M_CAND, 3, 3))[:, 0]
    h_pad = jnp.concatenate(
        [h_cand, jnp.full((K_PAD - M_CAND,), -jnp.inf, jnp.float32)])
    oidx_pad = jnp.concatenate(
        [oidx, (N + jnp.arange(K_PAD - M_CAND, dtype=jnp.int32))[:, None]])
    return h_pad, oidx_pad


def kernel(src, tgt, src_embedding, tgt_embedding):
    ps3 = src[0]
    pt3 = tgt[0]
    idx_s = _knn32(ps3)
    idx_t = _knn32(pt3)
    hs, os_col = _candidates(ps3, idx_s)
    ht, ot_col = _candidates(pt3, idx_t)

    ps = jnp.pad(ps3, ((0, 5), (0, 0)))
    ptg = jnp.pad(pt3, ((0, 5), (0, 0)))
    es = src_embedding[0]
    et = tgt_embedding[0]

    opts, optt, oes, oet, oit0, oit1 = _tail(
        hs[:, None], hs[None, :], os_col, os_col.reshape(1, K_PAD),
        ht[:, None], ht[None, :], ot_col, ot_col.reshape(1, K_PAD),
        ps, ptg, es, et)

    it0 = oit0[:N_SAMP, 0]
    it1 = oit1[:N_SAMP, 0]
    return (opts[None, :3, :N_SAMP], optt[None, :3, :N_SAMP],
            oes[None, :, :N_SAMP], oet[None, :, :N_SAMP], it0, it1)
